# Initial kernel scaffold; baseline (speedup 1.0000x reference)
import jax
import jax.numpy as jnp
from jax.experimental import pallas as pl


def kernel(x, edge_index, edge_weight, edge_attr, W0, b0, W1, b1, W2, b2):
    raise NotImplementedError("write your pallas kernel here")


# speedup vs baseline: 7.4415x; 7.4415x over previous
import dataclasses
import functools

import jax
import jax.numpy as jnp
from jax import lax
from jax.experimental import pallas as pl
from jax.experimental.pallas import tpu as pltpu
from jax.experimental.pallas import tpu_sc as plsc

NC = 2
NS = 16
NL = 16
CHUNK = 128

_MESH = functools.partial(
    plsc.VectorSubcoreMesh, core_axis_name="c", subcore_axis_name="s"
)


def _sc_params():
    cp = pltpu.CompilerParams()
    if "needs_layout_passes" in pltpu.CompilerParams.__dataclass_fields__:
        cp = dataclasses.replace(cp, needs_layout_passes=False)
    return cp


def _rsqrt16(d):
    i = plsc.bitcast(d, jnp.int32)
    i = jnp.int32(0x5F3759DF) - (i >> 1)
    y = plsc.bitcast(i, jnp.float32)
    for _ in range(3):
        y = y * (1.5 - 0.5 * d * y * y)
    return jnp.where(d > 0, y, 0.0)


def _splat16(vals_ref, e):
    return plsc.load_gather(vals_ref, [jnp.full((16,), e, jnp.int32)])


def _sc_deg(dst, ew, n_pad, e_pad):
    nh = n_pad // 2
    npc = n_pad // NS
    epw = e_pad // (NC * NS)
    nchunks = epw // CHUNK

    def body(dst_hbm, ew_hbm, degp_hbm, idx_v, ew_v, h2, hist_v, seg_v,
             acc_v, slab_sh):
        cid = lax.axis_index("c")
        sid = lax.axis_index("s")
        ebase = cid * (e_pad // NC) + sid * epw
        iota = lax.iota(jnp.int32, 16)
        zero16 = jnp.zeros((16,), jnp.float32)

        for p in range(2):
            lo = p * nh

            @pl.loop(0, nh // 16)
            def _(j):
                for r in range(16):
                    h2[r, pl.ds(j * 16, 16)] = zero16

            @pl.loop(0, nchunks)
            def _(j):
                base = ebase + j * CHUNK
                pltpu.sync_copy(dst_hbm.at[pl.ds(base, CHUNK)], idx_v)
                pltpu.sync_copy(ew_hbm.at[pl.ds(base, CHUNK)], ew_v)

                @pl.loop(0, CHUNK // 16)
                def _(k):
                    d16 = idx_v[pl.ds(k * 16, 16)] - lo
                    w16 = ew_v[pl.ds(k * 16, 16)]
                    msk = (d16 >= 0) & (d16 < nh)
                    dc = jnp.minimum(jnp.maximum(d16, 0), nh - 1)
                    plsc.addupdate_scatter(h2, [iota, dc], w16, mask=msk)

            @pl.loop(0, nh // 16)
            def _(j):
                a = zero16
                for r in range(16):
                    a = a + h2[r, pl.ds(j * 16, 16)]
                hist_v[pl.ds(lo + j * 16, 16)] = a

        pltpu.sync_copy(hist_v, slab_sh.at[sid])
        plsc.subcore_barrier()

        @pl.loop(0, npc // 16)
        def _(j):
            acc_v[pl.ds(j * 16, 16)] = zero16

        for r in range(NS):
            pltpu.sync_copy(slab_sh.at[r, pl.ds(sid * npc, npc)], seg_v)

            @pl.loop(0, npc // 16)
            def _(j):
                sl = pl.ds(j * 16, 16)
                acc_v[sl] = acc_v[sl] + seg_v[sl]

        pltpu.sync_copy(acc_v, degp_hbm.at[cid, pl.ds(sid * npc, npc)])

    return pl.kernel(
        body,
        out_type=jax.ShapeDtypeStruct((NC, n_pad), jnp.float32),
        mesh=_MESH(),
        compiler_params=_sc_params(),
        scratch_types=[
            pltpu.VMEM((CHUNK,), jnp.int32),
            pltpu.VMEM((CHUNK,), jnp.float32),
            pltpu.VMEM((16, n_pad // 2), jnp.float32),
            pltpu.VMEM((n_pad,), jnp.float32),
            pltpu.VMEM((n_pad // NS,), jnp.float32),
            pltpu.VMEM((n_pad // NS,), jnp.float32),
            pltpu.VMEM_SHARED((NS, n_pad), jnp.float32),
        ],
    )(dst, ew)


def _sc_norm(degp, src, dst, ew, n_pad, e_pad):
    epw = e_pad // (NC * NS)
    nchunks = epw // CHUNK

    def body(degp_hbm, src_hbm, dst_hbm, ew_hbm, norm_hbm, deg2_v, dis_v,
             s_v, d_v, w_v, n_v):
        cid = lax.axis_index("c")
        sid = lax.axis_index("s")
        pltpu.sync_copy(degp_hbm, deg2_v)

        @pl.loop(0, n_pad // 16)
        def _(i):
            a = deg2_v[0, pl.ds(i * 16, 16)]
            b = deg2_v[1, pl.ds(i * 16, 16)]
            dis_v[pl.ds(i * 16, 16)] = _rsqrt16(a + b)

        ebase = cid * (e_pad // NC) + sid * epw

        @pl.loop(0, nchunks)
        def _(j):
            base = ebase + j * CHUNK
            pltpu.sync_copy(src_hbm.at[pl.ds(base, CHUNK)], s_v)
            pltpu.sync_copy(dst_hbm.at[pl.ds(base, CHUNK)], d_v)
            pltpu.sync_copy(ew_hbm.at[pl.ds(base, CHUNK)], w_v)

            @pl.loop(0, CHUNK // 16)
            def _(k):
                s16 = s_v[pl.ds(k * 16, 16)]
                d16 = d_v[pl.ds(k * 16, 16)]
                a = plsc.load_gather(dis_v, [s16])
                b = plsc.load_gather(dis_v, [d16])
                n_v[pl.ds(k * 16, 16)] = a * w_v[pl.ds(k * 16, 16)] * b

            pltpu.sync_copy(n_v, norm_hbm.at[pl.ds(base, CHUNK)])

    return pl.kernel(
        body,
        out_type=jax.ShapeDtypeStruct((e_pad,), jnp.float32),
        mesh=_MESH(),
        compiler_params=_sc_params(),
        scratch_types=[
            pltpu.VMEM((NC, n_pad), jnp.float32),
            pltpu.VMEM((n_pad,), jnp.float32),
            pltpu.VMEM((CHUNK,), jnp.int32),
            pltpu.VMEM((CHUNK,), jnp.int32),
            pltpu.VMEM((CHUNK,), jnp.float32),
            pltpu.VMEM((CHUNK,), jnp.float32),
        ],
    )(degp, src, dst, ew)


def _sc_agg(ht, src, dst, norm, n_pad, e_pad):
    d = ht.shape[1]
    npc = n_pad // NS
    epw = e_pad // (NC * NS)
    nchunks = epw // CHUNK

    def body(ht_hbm, src_hbm, dst_hbm, norm_hbm, aggp_hbm, s_v, d_v, n_v,
             rows_v, acc_sh):
        cid = lax.axis_index("c")
        sid = lax.axis_index("s")
        row0 = sid * npc

        @pl.loop(0, CHUNK)
        def _(i):
            for k in range(d // 16):
                rows_v[i, pl.ds(k * 16, 16)] = jnp.zeros((16,), jnp.float32)

        for t in range(npc // CHUNK):
            pltpu.sync_copy(rows_v, acc_sh.at[pl.ds(row0 + t * CHUNK, CHUNK)])
        plsc.subcore_barrier()

        ebase = cid * (e_pad // NC) + sid * epw

        @pl.loop(0, nchunks)
        def _(j):
            base = ebase + j * CHUNK
            pltpu.sync_copy(src_hbm.at[pl.ds(base, CHUNK)], s_v)
            pltpu.sync_copy(dst_hbm.at[pl.ds(base, CHUNK)], d_v)
            pltpu.sync_copy(norm_hbm.at[pl.ds(base, CHUNK)], n_v)
            pltpu.sync_copy(ht_hbm.at[s_v], rows_v)

            @pl.loop(0, CHUNK)
            def _(e):
                nsplat = _splat16(n_v, e)
                for k in range(d // 16):
                    sl = pl.ds(k * 16, 16)
                    rows_v[e, sl] = rows_v[e, sl] * nsplat

            pltpu.sync_copy(rows_v, acc_sh.at[d_v], add=True)

        plsc.subcore_barrier()

        @pl.loop(0, npc // CHUNK)
        def _(t):
            pltpu.sync_copy(acc_sh.at[pl.ds(row0 + t * CHUNK, CHUNK)],
                            rows_v)
            pltpu.sync_copy(rows_v,
                            aggp_hbm.at[cid, pl.ds(row0 + t * CHUNK, CHUNK)])

    return pl.kernel(
        body,
        out_type=jax.ShapeDtypeStruct((NC, n_pad, d), jnp.float32),
        mesh=_MESH(),
        compiler_params=_sc_params(),
        scratch_types=[
            pltpu.VMEM((CHUNK,), jnp.int32),
            pltpu.VMEM((CHUNK,), jnp.int32),
            pltpu.VMEM((CHUNK,), jnp.float32),
            pltpu.VMEM((CHUNK, d), jnp.float32),
            pltpu.VMEM_SHARED((n_pad, d), jnp.float32),
        ],
    )(ht, src, dst, norm)


def _tc_in(x, w0, b0, w1, blk):
    n, d = x.shape

    def body(x_ref, w0_ref, b0_ref, w1_ref, h0_ref, ht1_ref):
        h0 = jnp.maximum(
            jnp.dot(x_ref[...], w0_ref[...],
                    preferred_element_type=jnp.float32) + b0_ref[...], 0.0)
        h0_ref[...] = h0
        ht1_ref[...] = jnp.dot(h0, w1_ref[...],
                               preferred_element_type=jnp.float32)

    return pl.pallas_call(
        body,
        grid=(n // blk,),
        in_specs=[
            pl.BlockSpec((blk, d), lambda i: (i, 0)),
            pl.BlockSpec((d, d), lambda i: (0, 0)),
            pl.BlockSpec((1, d), lambda i: (0, 0)),
            pl.BlockSpec((d, d), lambda i: (0, 0)),
        ],
        out_specs=[pl.BlockSpec((blk, d), lambda i: (i, 0))] * 2,
        out_shape=[jax.ShapeDtypeStruct((n, d), jnp.float32)] * 2,
    )(x, w0, b0.reshape(1, d), w1)


def _tc_mid(h0, aggp, b1, w2, blk):
    n, d = h0.shape

    def body(h0_ref, a0_ref, a1_ref, b1_ref, w2_ref, h1_ref, ht2_ref):
        g = jnp.maximum(a0_ref[0] + a1_ref[0] + b1_ref[...], 0.0)
        h1 = h0_ref[...] + g
        h1_ref[...] = h1
        ht2_ref[...] = jnp.dot(h1, w2_ref[...],
                               preferred_element_type=jnp.float32)

    return pl.pallas_call(
        body,
        grid=(n // blk,),
        in_specs=[
            pl.BlockSpec((blk, d), lambda i: (i, 0)),
            pl.BlockSpec((1, blk, d), lambda i: (0, i, 0)),
            pl.BlockSpec((1, blk, d), lambda i: (1, i, 0)),
            pl.BlockSpec((1, d), lambda i: (0, 0)),
            pl.BlockSpec((d, d), lambda i: (0, 0)),
        ],
        out_specs=[pl.BlockSpec((blk, d), lambda i: (i, 0))] * 2,
        out_shape=[jax.ShapeDtypeStruct((n, d), jnp.float32)] * 2,
    )(h0, aggp, aggp, b1.reshape(1, d), w2)


def _tc_out(h1, aggp, b2, blk):
    n, d = h1.shape

    def body(h1_ref, a0_ref, a1_ref, b2_ref, o_ref):
        g = jnp.maximum(a0_ref[0] + a1_ref[0] + b2_ref[...], 0.0)
        o_ref[...] = h1_ref[...] + g

    return pl.pallas_call(
        body,
        grid=(n // blk,),
        in_specs=[
            pl.BlockSpec((blk, d), lambda i: (i, 0)),
            pl.BlockSpec((1, blk, d), lambda i: (0, i, 0)),
            pl.BlockSpec((1, blk, d), lambda i: (1, i, 0)),
            pl.BlockSpec((1, d), lambda i: (0, 0)),
        ],
        out_specs=pl.BlockSpec((blk, d), lambda i: (i, 0)),
        out_shape=jax.ShapeDtypeStruct((n, d), jnp.float32),
    )(h1, aggp, aggp, b2.reshape(1, d))


def kernel(x, edge_index, edge_weight, edge_attr, W0, b0, W1, b1, W2, b2):
    del edge_attr
    n, d = x.shape
    e = edge_index.shape[1]

    src = edge_index[0].astype(jnp.int32)
    dst = edge_index[1].astype(jnp.int32)
    ew = edge_weight.astype(jnp.float32)

    e_pad = -(-e // (NC * NS * CHUNK)) * (NC * NS * CHUNK)
    npad = e_pad - e
    if npad:
        pidx = (jnp.arange(npad, dtype=jnp.int32) * 97) % n
        src = jnp.concatenate([src, pidx])
        dst = jnp.concatenate([dst, pidx])
        ew = jnp.concatenate([ew, jnp.zeros((npad,), jnp.float32)])

    n_pad = -(-n // (NS * NL)) * (NS * NL)

    blk = 1000 if n % 1000 == 0 else 8

    degp = _sc_deg(dst, ew, n_pad, e_pad)
    norm = _sc_norm(degp, src, dst, ew, n_pad, e_pad)

    h0, ht1 = _tc_in(x, W0, b0, W1, blk)
    agg1 = _sc_agg(ht1, src, dst, norm, n_pad, e_pad)
    h1, ht2 = _tc_mid(h0, agg1, b1, W2, blk)
    agg2 = _sc_agg(ht2, src, dst, norm, n_pad, e_pad)
    return _tc_out(h1, agg2, b2, blk)


# speedup vs baseline: 15.0567x; 2.0233x over previous
import dataclasses
import functools

import jax
import jax.numpy as jnp
from jax import lax
from jax.experimental import pallas as pl
from jax.experimental.pallas import tpu as pltpu
from jax.experimental.pallas import tpu_sc as plsc

NC = 2
NS = 16
NL = 16
CHUNK = 128
NBUF = 2

_MESH = functools.partial(
    plsc.VectorSubcoreMesh, core_axis_name="c", subcore_axis_name="s"
)


def _sc_params():
    cp = pltpu.CompilerParams()
    if "needs_layout_passes" in pltpu.CompilerParams.__dataclass_fields__:
        cp = dataclasses.replace(cp, needs_layout_passes=False)
    return cp


def _rsqrt16(d):
    i = plsc.bitcast(d, jnp.int32)
    i = jnp.int32(0x5F3759DF) - (i >> 1)
    y = plsc.bitcast(i, jnp.float32)
    for _ in range(3):
        y = y * (1.5 - 0.5 * d * y * y)
    return jnp.where(d > 0, y, 0.0)


def _splat16(vals_ref, e):
    return plsc.load_gather(vals_ref, [jnp.full((16,), e, jnp.int32)])


def _tree_add(vs):
    while len(vs) > 1:
        vs = [a + b for a, b in zip(vs[::2], vs[1::2])]
    return vs[0]


def _sc_deg(dst2, ew2, n_pad, e_pad):
    nh = n_pad // 4
    npc = n_pad // NS
    epw = e_pad // (NC * NS)
    nchunks = epw // CHUNK

    def body(dst_hbm, ew_hbm, degp_hbm, d2_v, w2_v, h2, hist_v, seg_v,
             acc_v, slab_sh):
        cid = lax.axis_index("c")
        sid = lax.axis_index("s")
        wid = cid * NS + sid
        iota = lax.iota(jnp.int32, 16)
        zero16 = jnp.zeros((16,), jnp.float32)

        pltpu.sync_copy(dst_hbm.at[pl.ds(wid * nchunks, nchunks)], d2_v)
        pltpu.sync_copy(ew_hbm.at[pl.ds(wid * nchunks, nchunks)], w2_v)

        for p in range(4):
            lo = p * nh

            @pl.loop(0, nh // 16)
            def _(j):
                for r in range(16):
                    h2[r, pl.ds(j * 16, 16)] = zero16

            @pl.loop(0, nchunks)
            def _(j):
                @pl.loop(0, CHUNK // 16)
                def _(k):
                    d16 = d2_v[j, pl.ds(k * 16, 16)] - lo
                    w16 = w2_v[j, pl.ds(k * 16, 16)]
                    msk = (d16 >= 0) & (d16 < nh)
                    dc = jnp.minimum(jnp.maximum(d16, 0), nh - 1)
                    plsc.addupdate_scatter(h2, [iota, dc], w16, mask=msk)

            @pl.loop(0, nh // 16)
            def _(j):
                vs = [h2[r, pl.ds(j * 16, 16)] for r in range(16)]
                hist_v[pl.ds(lo + j * 16, 16)] = _tree_add(vs)

        pltpu.sync_copy(hist_v, slab_sh.at[sid])
        plsc.subcore_barrier()

        @pl.loop(0, npc // 16)
        def _(j):
            acc_v[pl.ds(j * 16, 16)] = zero16

        for r in range(NS):
            pltpu.sync_copy(slab_sh.at[r, pl.ds(sid * npc, npc)], seg_v)

            @pl.loop(0, npc // 16)
            def _(j):
                sl = pl.ds(j * 16, 16)
                acc_v[sl] = acc_v[sl] + seg_v[sl]

        pltpu.sync_copy(acc_v, degp_hbm.at[cid, pl.ds(sid * npc, npc)])

    return pl.kernel(
        body,
        out_type=jax.ShapeDtypeStruct((NC, n_pad), jnp.float32),
        mesh=_MESH(),
        compiler_params=_sc_params(),
        scratch_types=[
            pltpu.VMEM((nchunks, CHUNK), jnp.int32),
            pltpu.VMEM((nchunks, CHUNK), jnp.float32),
            pltpu.VMEM((16, n_pad // 4), jnp.float32),
            pltpu.VMEM((n_pad,), jnp.float32),
            pltpu.VMEM((n_pad // NS,), jnp.float32),
            pltpu.VMEM((n_pad // NS,), jnp.float32),
            pltpu.VMEM_SHARED((NS, n_pad), jnp.float32),
        ],
    )(dst2, ew2)


def _sc_norm(degp, src2, dst2, ew2, n_pad, e_pad):
    epw = e_pad // (NC * NS)
    nchunks = epw // CHUNK

    def body(degp_hbm, src_hbm, dst_hbm, ew_hbm, norm_hbm, deg2_v, dis_v,
             s2_v, d2_v, w2_v, n_v):
        cid = lax.axis_index("c")
        sid = lax.axis_index("s")
        wid = cid * NS + sid
        pltpu.sync_copy(degp_hbm, deg2_v)
        pltpu.sync_copy(src_hbm.at[pl.ds(wid * nchunks, nchunks)], s2_v)
        pltpu.sync_copy(dst_hbm.at[pl.ds(wid * nchunks, nchunks)], d2_v)
        pltpu.sync_copy(ew_hbm.at[pl.ds(wid * nchunks, nchunks)], w2_v)

        @pl.loop(0, n_pad // 16)
        def _(i):
            a = deg2_v[0, pl.ds(i * 16, 16)]
            b = deg2_v[1, pl.ds(i * 16, 16)]
            dis_v[pl.ds(i * 16, 16)] = _rsqrt16(a + b)

        @pl.loop(0, nchunks)
        def _(j):
            @pl.loop(0, CHUNK // 16)
            def _(k):
                s16 = s2_v[j, pl.ds(k * 16, 16)]
                d16 = d2_v[j, pl.ds(k * 16, 16)]
                a = plsc.load_gather(dis_v, [s16])
                b = plsc.load_gather(dis_v, [d16])
                n_v[pl.ds(k * 16, 16)] = (
                    a * w2_v[j, pl.ds(k * 16, 16)] * b)

            pltpu.sync_copy(
                n_v, norm_hbm.at[pl.ds(wid * epw + j * CHUNK, CHUNK)])

    return pl.kernel(
        body,
        out_type=jax.ShapeDtypeStruct((e_pad,), jnp.float32),
        mesh=_MESH(),
        compiler_params=_sc_params(),
        scratch_types=[
            pltpu.VMEM((NC, n_pad), jnp.float32),
            pltpu.VMEM((n_pad,), jnp.float32),
            pltpu.VMEM((nchunks, CHUNK), jnp.int32),
            pltpu.VMEM((nchunks, CHUNK), jnp.int32),
            pltpu.VMEM((nchunks, CHUNK), jnp.float32),
            pltpu.VMEM((CHUNK,), jnp.float32),
        ],
    )(degp, src2, dst2, ew2)


def _sc_agg(ht, src2, dst2, norm, n_pad, e_pad):
    d = ht.shape[1]
    npc = n_pad // NS
    epw = e_pad // (NC * NS)
    nchunks = epw // CHUNK
    assert nchunks % NBUF == 0 and nchunks >= 2 * NBUF

    def body(ht_hbm, src_hbm, dst_hbm, norm_hbm, aggp_hbm, s_v, d_v,
             n_v, rows, acc_sh, gsems):
        cid = lax.axis_index("c")
        sid = lax.axis_index("s")
        wid = cid * NS + sid
        row0 = sid * npc
        ebase = wid * nchunks
        zero16 = jnp.zeros((16,), jnp.float32)

        @pl.loop(0, CHUNK)
        def _(i):
            for k in range(d // 16):
                rows[0][i, pl.ds(k * 16, 16)] = zero16

        for t in range(npc // CHUNK):
            pltpu.sync_copy(rows[0],
                            acc_sh.at[pl.ds(row0 + t * CHUNK, CHUNK)])
        plsc.subcore_barrier()

        def fetch(j, b):
            pltpu.async_copy(src_hbm.at[ebase + j], s_v[b], gsems[b])
            pltpu.async_copy(dst_hbm.at[ebase + j], d_v[b], gsems[b])
            pltpu.async_copy(
                norm_hbm.at[pl.ds((ebase + j) * CHUNK, CHUNK)], n_v[b],
                gsems[b])
            pltpu.make_async_copy(src_hbm.at[ebase + j], s_v[b],
                                  gsems[b]).wait()
            pltpu.async_copy(ht_hbm.at[s_v[b]], rows[b], gsems[b])

        def wait_fetch(j, b):
            pltpu.make_async_copy(dst_hbm.at[ebase + j], d_v[b],
                                  gsems[b]).wait()
            pltpu.make_async_copy(
                norm_hbm.at[pl.ds((ebase + j) * CHUNK, CHUNK)], n_v[b],
                gsems[b]).wait()
            pltpu.make_async_copy(ht_hbm.at[s_v[b]], rows[b],
                                  gsems[b]).wait()

        for b in range(NBUF):
            fetch(b, b)

        @pl.loop(0, nchunks, step=NBUF)
        def _(j0):
            for b in range(NBUF):
                j = j0 + b
                wait_fetch(j, b)

                @pl.loop(0, CHUNK)
                def _(e):
                    nsplat = _splat16(n_v[b], e)
                    for k in range(d // 16):
                        sl = pl.ds(k * 16, 16)
                        rows[b][e, sl] = rows[b][e, sl] * nsplat

                pltpu.sync_copy(rows[b], acc_sh.at[d_v[b]], add=True)

                @pl.when(j + NBUF < nchunks)
                def _():
                    fetch(j + NBUF, b)

        plsc.subcore_barrier()

        @pl.loop(0, npc // CHUNK)
        def _(t):
            pltpu.sync_copy(acc_sh.at[pl.ds(row0 + t * CHUNK, CHUNK)],
                            rows[0])
            pltpu.sync_copy(
                rows[0], aggp_hbm.at[cid, pl.ds(row0 + t * CHUNK, CHUNK)])

    return pl.kernel(
        body,
        out_type=jax.ShapeDtypeStruct((NC, n_pad, d), jnp.float32),
        mesh=_MESH(),
        compiler_params=_sc_params(),
        scratch_types=[
            [pltpu.VMEM((CHUNK,), jnp.int32) for _ in range(NBUF)],
            [pltpu.VMEM((CHUNK,), jnp.int32) for _ in range(NBUF)],
            [pltpu.VMEM((CHUNK,), jnp.float32) for _ in range(NBUF)],
            [pltpu.VMEM((CHUNK, d), jnp.float32) for _ in range(NBUF)],
            pltpu.VMEM_SHARED((n_pad, d), jnp.float32),
            [pltpu.SemaphoreType.DMA for _ in range(NBUF)],
        ],
    )(ht, src2, dst2, norm)


def _tc_in(x, w0, b0, w1, blk):
    n, d = x.shape

    def body(x_ref, w0_ref, b0_ref, w1_ref, h0_ref, ht1_ref):
        h0 = jnp.maximum(
            jnp.dot(x_ref[...], w0_ref[...],
                    preferred_element_type=jnp.float32) + b0_ref[...], 0.0)
        h0_ref[...] = h0
        ht1_ref[...] = jnp.dot(h0, w1_ref[...],
                               preferred_element_type=jnp.float32)

    return pl.pallas_call(
        body,
        grid=(n // blk,),
        in_specs=[
            pl.BlockSpec((blk, d), lambda i: (i, 0)),
            pl.BlockSpec((d, d), lambda i: (0, 0)),
            pl.BlockSpec((1, d), lambda i: (0, 0)),
            pl.BlockSpec((d, d), lambda i: (0, 0)),
        ],
        out_specs=[pl.BlockSpec((blk, d), lambda i: (i, 0))] * 2,
        out_shape=[jax.ShapeDtypeStruct((n, d), jnp.float32)] * 2,
    )(x, w0, b0.reshape(1, d), w1)


def _tc_mid(h0, aggp, b1, w2, blk):
    n, d = h0.shape

    def body(h0_ref, a0_ref, a1_ref, b1_ref, w2_ref, h1_ref, ht2_ref):
        g = jnp.maximum(a0_ref[0] + a1_ref[0] + b1_ref[...], 0.0)
        h1 = h0_ref[...] + g
        h1_ref[...] = h1
        ht2_ref[...] = jnp.dot(h1, w2_ref[...],
                               preferred_element_type=jnp.float32)

    return pl.pallas_call(
        body,
        grid=(n // blk,),
        in_specs=[
            pl.BlockSpec((blk, d), lambda i: (i, 0)),
            pl.BlockSpec((1, blk, d), lambda i: (0, i, 0)),
            pl.BlockSpec((1, blk, d), lambda i: (1, i, 0)),
            pl.BlockSpec((1, d), lambda i: (0, 0)),
            pl.BlockSpec((d, d), lambda i: (0, 0)),
        ],
        out_specs=[pl.BlockSpec((blk, d), lambda i: (i, 0))] * 2,
        out_shape=[jax.ShapeDtypeStruct((n, d), jnp.float32)] * 2,
    )(h0, aggp, aggp, b1.reshape(1, d), w2)


def _tc_out(h1, aggp, b2, blk):
    n, d = h1.shape

    def body(h1_ref, a0_ref, a1_ref, b2_ref, o_ref):
        g = jnp.maximum(a0_ref[0] + a1_ref[0] + b2_ref[...], 0.0)
        o_ref[...] = h1_ref[...] + g

    return pl.pallas_call(
        body,
        grid=(n // blk,),
        in_specs=[
            pl.BlockSpec((blk, d), lambda i: (i, 0)),
            pl.BlockSpec((1, blk, d), lambda i: (0, i, 0)),
            pl.BlockSpec((1, blk, d), lambda i: (1, i, 0)),
            pl.BlockSpec((1, d), lambda i: (0, 0)),
        ],
        out_specs=pl.BlockSpec((blk, d), lambda i: (i, 0)),
        out_shape=jax.ShapeDtypeStruct((n, d), jnp.float32),
    )(h1, aggp, aggp, b2.reshape(1, d))


def kernel(x, edge_index, edge_weight, edge_attr, W0, b0, W1, b1, W2, b2):
    del edge_attr
    n, d = x.shape
    e = edge_index.shape[1]

    src = edge_index[0].astype(jnp.int32)
    dst = edge_index[1].astype(jnp.int32)
    ew = edge_weight.astype(jnp.float32)

    grain = NC * NS * NBUF * CHUNK
    e_pad = -(-e // grain) * grain
    npad = e_pad - e
    if npad:
        pidx = (jnp.arange(npad, dtype=jnp.int32) * 97) % n
        src = jnp.concatenate([src, pidx])
        dst = jnp.concatenate([dst, pidx])
        ew = jnp.concatenate([ew, jnp.zeros((npad,), jnp.float32)])

    src2 = src.reshape(e_pad // CHUNK, CHUNK)
    dst2 = dst.reshape(e_pad // CHUNK, CHUNK)
    ew2 = ew.reshape(e_pad // CHUNK, CHUNK)

    n_pad = -(-n // (NS * NL)) * (NS * NL)

    blk = 1000 if n % 1000 == 0 else 8

    degp = _sc_deg(dst2, ew2, n_pad, e_pad)
    norm = _sc_norm(degp, src2, dst2, ew2, n_pad, e_pad)

    h0, ht1 = _tc_in(x, W0, b0, W1, blk)
    agg1 = _sc_agg(ht1, src2, dst2, norm, n_pad, e_pad)
    h1, ht2 = _tc_mid(h0, agg1, b1, W2, blk)
    agg2 = _sc_agg(ht2, src2, dst2, norm, n_pad, e_pad)
    return _tc_out(h1, agg2, b2, blk)


# speedup vs baseline: 19.9983x; 1.3282x over previous
import dataclasses
import functools

import jax
import jax.numpy as jnp
from jax import lax
from jax.experimental import pallas as pl
from jax.experimental.pallas import tpu as pltpu
from jax.experimental.pallas import tpu_sc as plsc

NC = 2
NS = 16
NL = 16
CHUNK = 80
NBUF = 3

_MESH = functools.partial(
    plsc.VectorSubcoreMesh, core_axis_name="c", subcore_axis_name="s"
)


def _sc_params():
    cp = pltpu.CompilerParams()
    if "needs_layout_passes" in pltpu.CompilerParams.__dataclass_fields__:
        cp = dataclasses.replace(cp, needs_layout_passes=False)
    return cp


def _rsqrt16(d):
    i = plsc.bitcast(d, jnp.int32)
    i = jnp.int32(0x5F3759DF) - (i >> 1)
    y = plsc.bitcast(i, jnp.float32)
    for _ in range(3):
        y = y * (1.5 - 0.5 * d * y * y)
    return jnp.where(d > 0, y, 0.0)


def _splat16(vals_ref, e):
    return plsc.load_gather(vals_ref, [jnp.full((16,), e, jnp.int32)])


def _tree_add(vs):
    while len(vs) > 1:
        vs = [a + b for a, b in zip(vs[::2], vs[1::2])]
    return vs[0]


def _sc_deg(dst, ew, n_pad, e_pad):
    nh = n_pad // 4
    npc = n_pad // NS
    epw = e_pad // (NC * NS)
    nchunks = epw // CHUNK

    def body(dst_hbm, ew_hbm, degp_hbm, d2_v, w2_v, h2, hist_v, seg_v,
             acc_v, slab_sh):
        cid = lax.axis_index("c")
        sid = lax.axis_index("s")
        wid = cid * NS + sid
        iota = lax.iota(jnp.int32, 16)
        zero16 = jnp.zeros((16,), jnp.float32)

        pltpu.sync_copy(dst_hbm.at[pl.ds(wid * epw, epw)], d2_v)
        pltpu.sync_copy(ew_hbm.at[pl.ds(wid * epw, epw)], w2_v)

        for p in range(4):
            lo = p * nh

            @pl.loop(0, nh // 16)
            def _(j):
                for r in range(16):
                    h2[r, pl.ds(j * 16, 16)] = zero16

            @pl.loop(0, epw // 16)
            def _(k):
                d16 = d2_v[pl.ds(k * 16, 16)] - lo
                w16 = w2_v[pl.ds(k * 16, 16)]
                msk = (d16 >= 0) & (d16 < nh)
                dc = jnp.minimum(jnp.maximum(d16, 0), nh - 1)
                plsc.addupdate_scatter(h2, [iota, dc], w16, mask=msk)

            @pl.loop(0, nh // 16)
            def _(j):
                vs = [h2[r, pl.ds(j * 16, 16)] for r in range(16)]
                hist_v[pl.ds(lo + j * 16, 16)] = _tree_add(vs)

        pltpu.sync_copy(hist_v, slab_sh.at[sid])
        plsc.subcore_barrier()

        @pl.loop(0, npc // 16)
        def _(j):
            acc_v[pl.ds(j * 16, 16)] = zero16

        for r in range(NS):
            pltpu.sync_copy(slab_sh.at[r, pl.ds(sid * npc, npc)], seg_v)

            @pl.loop(0, npc // 16)
            def _(j):
                sl = pl.ds(j * 16, 16)
                acc_v[sl] = acc_v[sl] + seg_v[sl]

        pltpu.sync_copy(acc_v, degp_hbm.at[cid, pl.ds(sid * npc, npc)])

    return pl.kernel(
        body,
        out_type=jax.ShapeDtypeStruct((NC, n_pad), jnp.float32),
        mesh=_MESH(),
        compiler_params=_sc_params(),
        scratch_types=[
            pltpu.VMEM((epw,), jnp.int32),
            pltpu.VMEM((epw,), jnp.float32),
            pltpu.VMEM((16, n_pad // 4), jnp.float32),
            pltpu.VMEM((n_pad,), jnp.float32),
            pltpu.VMEM((n_pad // NS,), jnp.float32),
            pltpu.VMEM((n_pad // NS,), jnp.float32),
            pltpu.VMEM_SHARED((NS, n_pad), jnp.float32),
        ],
    )(dst, ew)


def _sc_norm(degp, src, dst, ew, n_pad, e_pad):
    epw = e_pad // (NC * NS)
    nchunks = epw // CHUNK

    def body(degp_hbm, src_hbm, dst_hbm, ew_hbm, norm_hbm, deg2_v, dis_v,
             s2_v, d2_v, w2_v, n_v):
        cid = lax.axis_index("c")
        sid = lax.axis_index("s")
        wid = cid * NS + sid
        pltpu.sync_copy(degp_hbm, deg2_v)
        pltpu.sync_copy(src_hbm.at[pl.ds(wid * epw, epw)], s2_v)
        pltpu.sync_copy(dst_hbm.at[pl.ds(wid * epw, epw)], d2_v)
        pltpu.sync_copy(ew_hbm.at[pl.ds(wid * epw, epw)], w2_v)

        @pl.loop(0, n_pad // 16)
        def _(i):
            a = deg2_v[0, pl.ds(i * 16, 16)]
            b = deg2_v[1, pl.ds(i * 16, 16)]
            dis_v[pl.ds(i * 16, 16)] = _rsqrt16(a + b)

        @pl.loop(0, epw // 16)
        def _(k):
            sl = pl.ds(k * 16, 16)
            a = plsc.load_gather(dis_v, [s2_v[sl]])
            b = plsc.load_gather(dis_v, [d2_v[sl]])
            n_v[sl] = a * w2_v[sl] * b

        pltpu.sync_copy(n_v, norm_hbm.at[pl.ds(wid * epw, epw)])

    return pl.kernel(
        body,
        out_type=jax.ShapeDtypeStruct((e_pad,), jnp.float32),
        mesh=_MESH(),
        compiler_params=_sc_params(),
        scratch_types=[
            pltpu.VMEM((NC, n_pad), jnp.float32),
            pltpu.VMEM((n_pad,), jnp.float32),
            pltpu.VMEM((epw,), jnp.int32),
            pltpu.VMEM((epw,), jnp.int32),
            pltpu.VMEM((epw,), jnp.float32),
            pltpu.VMEM((epw,), jnp.float32),
        ],
    )(degp, src, dst, ew)


def _sc_agg(ht, src, dst, norm, n_pad, e_pad):
    d = ht.shape[1]
    npc = n_pad // NS
    epw = e_pad // (NC * NS)
    nchunks = epw // CHUNK
    assert nchunks % NBUF == 0 and nchunks >= 2 * NBUF

    def body(ht_hbm, src_hbm, dst_hbm, norm_hbm, aggp_hbm, s2_v, d_v,
             n_v, rows, acc_sh, gsems, ssems):
        cid = lax.axis_index("c")
        sid = lax.axis_index("s")
        wid = cid * NS + sid
        row0 = sid * npc
        ebase = wid * nchunks
        zero16 = jnp.zeros((16,), jnp.float32)

        @pl.loop(0, CHUNK)
        def _(i):
            for k in range(d // 16):
                rows[0][i, pl.ds(k * 16, 16)] = zero16

        for t in range(npc // CHUNK):
            pltpu.sync_copy(rows[0],
                            acc_sh.at[pl.ds(row0 + t * CHUNK, CHUNK)])
        plsc.subcore_barrier()

        pltpu.sync_copy(src_hbm.at[pl.ds(wid * epw, epw)], s2_v)

        def fetch(j, b):
            pltpu.async_copy(
                dst_hbm.at[pl.ds((ebase + j) * CHUNK, CHUNK)], d_v[b],
                gsems[b])
            pltpu.async_copy(
                norm_hbm.at[pl.ds((ebase + j) * CHUNK, CHUNK)], n_v[b],
                gsems[b])
            pltpu.async_copy(ht_hbm.at[s2_v.at[pl.ds(j * CHUNK, CHUNK)]],
                             rows[b], gsems[b])

        def wait_fetch(j, b):
            pltpu.make_async_copy(
                dst_hbm.at[pl.ds((ebase + j) * CHUNK, CHUNK)], d_v[b],
                gsems[b]).wait()
            pltpu.make_async_copy(
                norm_hbm.at[pl.ds((ebase + j) * CHUNK, CHUNK)], n_v[b],
                gsems[b]).wait()
            pltpu.make_async_copy(
                ht_hbm.at[s2_v.at[pl.ds(j * CHUNK, CHUNK)]], rows[b],
                gsems[b]).wait()

        def wait_scat(b):
            pltpu.make_async_copy(rows[b], acc_sh.at[d_v[b]],
                                  ssems[b]).wait()

        fetch(0, 0)
        fetch(1, 1)

        @pl.loop(0, nchunks, step=NBUF)
        def _(j0):
            for b in range(NBUF):
                j = j0 + b
                wait_fetch(j, b)

                @pl.loop(0, CHUNK)
                def _(e):
                    nsplat = _splat16(n_v[b], e)
                    for k in range(d // 16):
                        sl = pl.ds(k * 16, 16)
                        rows[b][e, sl] = rows[b][e, sl] * nsplat

                pltpu.async_copy(rows[b], acc_sh.at[d_v[b]], ssems[b],
                                 add=True)

                b2 = (b + 2) % NBUF

                @pl.when(j + 2 < nchunks)
                def _():
                    @pl.when(j >= 1)
                    def _():
                        wait_scat(b2)

                    fetch(j + 2, b2)

        wait_scat((nchunks - 3) % NBUF)
        wait_scat((nchunks - 2) % NBUF)
        wait_scat((nchunks - 1) % NBUF)
        plsc.subcore_barrier()

        @pl.loop(0, npc // CHUNK)
        def _(t):
            pltpu.sync_copy(acc_sh.at[pl.ds(row0 + t * CHUNK, CHUNK)],
                            rows[0])
            pltpu.sync_copy(
                rows[0], aggp_hbm.at[cid, pl.ds(row0 + t * CHUNK, CHUNK)])

    return pl.kernel(
        body,
        out_type=jax.ShapeDtypeStruct((NC, n_pad, d), jnp.float32),
        mesh=_MESH(),
        compiler_params=_sc_params(),
        scratch_types=[
            pltpu.VMEM((epw,), jnp.int32),
            [pltpu.VMEM((CHUNK,), jnp.int32) for _ in range(NBUF)],
            [pltpu.VMEM((CHUNK,), jnp.float32) for _ in range(NBUF)],
            [pltpu.VMEM((CHUNK, d), jnp.float32) for _ in range(NBUF)],
            pltpu.VMEM_SHARED((n_pad, d), jnp.float32),
            [pltpu.SemaphoreType.DMA for _ in range(NBUF)],
            [pltpu.SemaphoreType.DMA for _ in range(NBUF)],
        ],
    )(ht, src, dst, norm)


def _tc_in(x, w0, b0, w1, blk):
    n, d = x.shape

    def body(x_ref, w0_ref, b0_ref, w1_ref, h0_ref, ht1_ref):
        h0 = jnp.maximum(
            jnp.dot(x_ref[...], w0_ref[...],
                    preferred_element_type=jnp.float32) + b0_ref[...], 0.0)
        h0_ref[...] = h0
        ht1_ref[...] = jnp.dot(h0, w1_ref[...],
                               preferred_element_type=jnp.float32)

    return pl.pallas_call(
        body,
        grid=(n // blk,),
        in_specs=[
            pl.BlockSpec((blk, d), lambda i: (i, 0)),
            pl.BlockSpec((d, d), lambda i: (0, 0)),
            pl.BlockSpec((1, d), lambda i: (0, 0)),
            pl.BlockSpec((d, d), lambda i: (0, 0)),
        ],
        out_specs=[pl.BlockSpec((blk, d), lambda i: (i, 0))] * 2,
        out_shape=[jax.ShapeDtypeStruct((n, d), jnp.float32)] * 2,
    )(x, w0, b0.reshape(1, d), w1)


def _tc_mid(h0, aggp, b1, w2, blk):
    n, d = h0.shape

    def body(h0_ref, a0_ref, a1_ref, b1_ref, w2_ref, h1_ref, ht2_ref):
        g = jnp.maximum(a0_ref[0] + a1_ref[0] + b1_ref[...], 0.0)
        h1 = h0_ref[...] + g
        h1_ref[...] = h1
        ht2_ref[...] = jnp.dot(h1, w2_ref[...],
                               preferred_element_type=jnp.float32)

    return pl.pallas_call(
        body,
        grid=(n // blk,),
        in_specs=[
            pl.BlockSpec((blk, d), lambda i: (i, 0)),
            pl.BlockSpec((1, blk, d), lambda i: (0, i, 0)),
            pl.BlockSpec((1, blk, d), lambda i: (1, i, 0)),
            pl.BlockSpec((1, d), lambda i: (0, 0)),
            pl.BlockSpec((d, d), lambda i: (0, 0)),
        ],
        out_specs=[pl.BlockSpec((blk, d), lambda i: (i, 0))] * 2,
        out_shape=[jax.ShapeDtypeStruct((n, d), jnp.float32)] * 2,
    )(h0, aggp, aggp, b1.reshape(1, d), w2)


def _tc_out(h1, aggp, b2, blk):
    n, d = h1.shape

    def body(h1_ref, a0_ref, a1_ref, b2_ref, o_ref):
        g = jnp.maximum(a0_ref[0] + a1_ref[0] + b2_ref[...], 0.0)
        o_ref[...] = h1_ref[...] + g

    return pl.pallas_call(
        body,
        grid=(n // blk,),
        in_specs=[
            pl.BlockSpec((blk, d), lambda i: (i, 0)),
            pl.BlockSpec((1, blk, d), lambda i: (0, i, 0)),
            pl.BlockSpec((1, blk, d), lambda i: (1, i, 0)),
            pl.BlockSpec((1, d), lambda i: (0, 0)),
        ],
        out_specs=pl.BlockSpec((blk, d), lambda i: (i, 0)),
        out_shape=jax.ShapeDtypeStruct((n, d), jnp.float32),
    )(h1, aggp, aggp, b2.reshape(1, d))


def kernel(x, edge_index, edge_weight, edge_attr, W0, b0, W1, b1, W2, b2):
    del edge_attr
    n, d = x.shape
    e = edge_index.shape[1]

    src = edge_index[0].astype(jnp.int32)
    dst = edge_index[1].astype(jnp.int32)
    ew = edge_weight.astype(jnp.float32)

    grain = NC * NS * NBUF * CHUNK
    e_pad = -(-e // grain) * grain
    npad = e_pad - e
    if npad:
        pidx = (jnp.arange(npad, dtype=jnp.int32) * 97) % n
        src = jnp.concatenate([src, pidx])
        dst = jnp.concatenate([dst, pidx])
        ew = jnp.concatenate([ew, jnp.zeros((npad,), jnp.float32)])

    n_pad = -(-n // (NS * NL)) * (NS * NL)

    blk = 1000 if n % 1000 == 0 else 8

    degp = _sc_deg(dst, ew, n_pad, e_pad)
    norm = _sc_norm(degp, src, dst, ew, n_pad, e_pad)

    h0, ht1 = _tc_in(x, W0, b0, W1, blk)
    agg1 = _sc_agg(ht1, src, dst, norm, n_pad, e_pad)
    h1, ht2 = _tc_mid(h0, agg1, b1, W2, blk)
    agg2 = _sc_agg(ht2, src, dst, norm, n_pad, e_pad)
    return _tc_out(h1, agg2, b2, blk)


# speedup vs baseline: 22.8643x; 1.1433x over previous
import dataclasses
import functools

import jax
import jax.numpy as jnp
from jax import lax
from jax.experimental import pallas as pl
from jax.experimental.pallas import tpu as pltpu
from jax.experimental.pallas import tpu_sc as plsc

NC = 2
NS = 16
NL = 16
CHUNK = 80
NBUF = 3

_MESH = functools.partial(
    plsc.VectorSubcoreMesh, core_axis_name="c", subcore_axis_name="s"
)


def _sc_params():
    cp = pltpu.CompilerParams()
    if "needs_layout_passes" in pltpu.CompilerParams.__dataclass_fields__:
        cp = dataclasses.replace(cp, needs_layout_passes=False)
    return cp


def _rsqrt16(d):
    i = plsc.bitcast(d, jnp.int32)
    i = jnp.int32(0x5F3759DF) - (i >> 1)
    y = plsc.bitcast(i, jnp.float32)
    for _ in range(3):
        y = y * (1.5 - 0.5 * d * y * y)
    return jnp.where(d > 0, y, 0.0)


def _splat16(vals_ref, e):
    return plsc.load_gather(vals_ref, [jnp.full((16,), e, jnp.int32)])


def _tree_add(vs):
    while len(vs) > 1:
        vs = [a + b for a, b in zip(vs[::2], vs[1::2])]
    return vs[0]


def _sc_deg(dst, ew, n_pad, e_pad):
    nh = n_pad // 2
    npc = n_pad // NS
    epw = e_pad // (NC * NS)
    nchunks = epw // CHUNK

    def body(dst_hbm, ew_hbm, degp_hbm, d2_v, w2_v, h2, hist_v, seg_v,
             acc_v, slab_sh):
        cid = lax.axis_index("c")
        sid = lax.axis_index("s")
        wid = cid * NS + sid
        iota = lax.iota(jnp.int32, 16)
        zero16 = jnp.zeros((16,), jnp.float32)

        pltpu.sync_copy(dst_hbm.at[pl.ds(wid * epw, epw)], d2_v)
        pltpu.sync_copy(ew_hbm.at[pl.ds(wid * epw, epw)], w2_v)

        for p in range(2):
            lo = p * nh

            @pl.loop(0, nh // 16)
            def _(j):
                for r in range(16):
                    h2[r, pl.ds(j * 16, 16)] = zero16

            @pl.loop(0, epw // 16)
            def _(k):
                d16 = d2_v[pl.ds(k * 16, 16)] - lo
                w16 = w2_v[pl.ds(k * 16, 16)]
                msk = (d16 >= 0) & (d16 < nh)
                dc = jnp.minimum(jnp.maximum(d16, 0), nh - 1)
                plsc.addupdate_scatter(h2, [iota, dc], w16, mask=msk)

            @pl.loop(0, nh // 16)
            def _(j):
                vs = [h2[r, pl.ds(j * 16, 16)] for r in range(16)]
                hist_v[pl.ds(lo + j * 16, 16)] = _tree_add(vs)

        pltpu.sync_copy(hist_v, slab_sh.at[sid])
        plsc.subcore_barrier()

        @pl.loop(0, npc // 16)
        def _(j):
            acc_v[pl.ds(j * 16, 16)] = zero16

        for r in range(NS):
            pltpu.sync_copy(slab_sh.at[r, pl.ds(sid * npc, npc)], seg_v)

            @pl.loop(0, npc // 16)
            def _(j):
                sl = pl.ds(j * 16, 16)
                acc_v[sl] = acc_v[sl] + seg_v[sl]

        pltpu.sync_copy(acc_v, degp_hbm.at[cid, pl.ds(sid * npc, npc)])

    return pl.kernel(
        body,
        out_type=jax.ShapeDtypeStruct((NC, n_pad), jnp.float32),
        mesh=_MESH(),
        compiler_params=_sc_params(),
        scratch_types=[
            pltpu.VMEM((epw,), jnp.int32),
            pltpu.VMEM((epw,), jnp.float32),
            pltpu.VMEM((16, n_pad // 2), jnp.float32),
            pltpu.VMEM((n_pad,), jnp.float32),
            pltpu.VMEM((n_pad // NS,), jnp.float32),
            pltpu.VMEM((n_pad // NS,), jnp.float32),
            pltpu.VMEM_SHARED((NS, n_pad), jnp.float32),
        ],
    )(dst, ew)


def _sc_norm(degp, src, dst, ew, n_pad, e_pad):
    epw = e_pad // (NC * NS)
    nchunks = epw // CHUNK

    def body(degp_hbm, src_hbm, dst_hbm, ew_hbm, norm_hbm, deg2_v, dis_v,
             s2_v, d2_v, w2_v, n_v):
        cid = lax.axis_index("c")
        sid = lax.axis_index("s")
        wid = cid * NS + sid
        pltpu.sync_copy(degp_hbm, deg2_v)
        pltpu.sync_copy(src_hbm.at[pl.ds(wid * epw, epw)], s2_v)
        pltpu.sync_copy(dst_hbm.at[pl.ds(wid * epw, epw)], d2_v)
        pltpu.sync_copy(ew_hbm.at[pl.ds(wid * epw, epw)], w2_v)

        @pl.loop(0, n_pad // 16)
        def _(i):
            a = deg2_v[0, pl.ds(i * 16, 16)]
            b = deg2_v[1, pl.ds(i * 16, 16)]
            dis_v[pl.ds(i * 16, 16)] = _rsqrt16(a + b)

        @pl.loop(0, epw // 16)
        def _(k):
            sl = pl.ds(k * 16, 16)
            a = plsc.load_gather(dis_v, [s2_v[sl]])
            b = plsc.load_gather(dis_v, [d2_v[sl]])
            n_v[sl] = a * w2_v[sl] * b

        pltpu.sync_copy(n_v, norm_hbm.at[pl.ds(wid * epw, epw)])

    return pl.kernel(
        body,
        out_type=jax.ShapeDtypeStruct((e_pad,), jnp.float32),
        mesh=_MESH(),
        compiler_params=_sc_params(),
        scratch_types=[
            pltpu.VMEM((NC, n_pad), jnp.float32),
            pltpu.VMEM((n_pad,), jnp.float32),
            pltpu.VMEM((epw,), jnp.int32),
            pltpu.VMEM((epw,), jnp.int32),
            pltpu.VMEM((epw,), jnp.float32),
            pltpu.VMEM((epw,), jnp.float32),
        ],
    )(degp, src, dst, ew)


def _sc_agg(ht, src, dst, norm, n_pad, e_pad):
    d = ht.shape[1]
    npc = n_pad // NS
    epw = e_pad // (NC * NS)
    nchunks = epw // CHUNK
    assert nchunks % NBUF == 0 and nchunks >= 2 * NBUF

    def body(ht_hbm, src_hbm, dst_hbm, norm_hbm, aggp_hbm, s2_v, d_v,
             n_v, rows, acc_sh, gsems, ssems):
        cid = lax.axis_index("c")
        sid = lax.axis_index("s")
        wid = cid * NS + sid
        row0 = sid * npc
        ebase = wid * nchunks
        zero16 = jnp.zeros((16,), jnp.float32)

        @pl.loop(0, CHUNK)
        def _(i):
            for k in range(d // 16):
                rows[0][i, pl.ds(k * 16, 16)] = zero16

        for t in range(npc // CHUNK):
            pltpu.sync_copy(rows[0],
                            acc_sh.at[pl.ds(row0 + t * CHUNK, CHUNK)])
        plsc.subcore_barrier()

        pltpu.sync_copy(src_hbm.at[pl.ds(wid * epw, epw)], s2_v)

        def fetch(j, b):
            pltpu.async_copy(
                dst_hbm.at[pl.ds((ebase + j) * CHUNK, CHUNK)], d_v[b],
                gsems[b])
            pltpu.async_copy(
                norm_hbm.at[pl.ds((ebase + j) * CHUNK, CHUNK)], n_v[b],
                gsems[b])
            pltpu.async_copy(ht_hbm.at[s2_v.at[pl.ds(j * CHUNK, CHUNK)]],
                             rows[b], gsems[b])

        def wait_fetch(j, b):
            pltpu.make_async_copy(
                dst_hbm.at[pl.ds((ebase + j) * CHUNK, CHUNK)], d_v[b],
                gsems[b]).wait()
            pltpu.make_async_copy(
                norm_hbm.at[pl.ds((ebase + j) * CHUNK, CHUNK)], n_v[b],
                gsems[b]).wait()
            pltpu.make_async_copy(
                ht_hbm.at[s2_v.at[pl.ds(j * CHUNK, CHUNK)]], rows[b],
                gsems[b]).wait()

        def wait_scat(b):
            pltpu.make_async_copy(rows[b], acc_sh.at[d_v[b]],
                                  ssems[b]).wait()

        fetch(0, 0)
        fetch(1, 1)

        @pl.loop(0, nchunks, step=NBUF)
        def _(j0):
            for b in range(NBUF):
                j = j0 + b
                wait_fetch(j, b)

                @plsc.parallel_loop(0, CHUNK, 1, unroll=2)
                def _(e):
                    nsplat = _splat16(n_v[b], e)
                    for k in range(d // 16):
                        sl = pl.ds(k * 16, 16)
                        rows[b][e, sl] = rows[b][e, sl] * nsplat

                pltpu.async_copy(rows[b], acc_sh.at[d_v[b]], ssems[b],
                                 add=True)

                b2 = (b + 2) % NBUF

                @pl.when(j + 2 < nchunks)
                def _():
                    @pl.when(j >= 1)
                    def _():
                        wait_scat(b2)

                    fetch(j + 2, b2)

        wait_scat((nchunks - 3) % NBUF)
        wait_scat((nchunks - 2) % NBUF)
        wait_scat((nchunks - 1) % NBUF)
        plsc.subcore_barrier()

        @pl.loop(0, npc // CHUNK)
        def _(t):
            pltpu.sync_copy(acc_sh.at[pl.ds(row0 + t * CHUNK, CHUNK)],
                            rows[0])
            pltpu.sync_copy(
                rows[0], aggp_hbm.at[cid, pl.ds(row0 + t * CHUNK, CHUNK)])

    return pl.kernel(
        body,
        out_type=jax.ShapeDtypeStruct((NC, n_pad, d), jnp.float32),
        mesh=_MESH(),
        compiler_params=_sc_params(),
        scratch_types=[
            pltpu.VMEM((epw,), jnp.int32),
            [pltpu.VMEM((CHUNK,), jnp.int32) for _ in range(NBUF)],
            [pltpu.VMEM((CHUNK,), jnp.float32) for _ in range(NBUF)],
            [pltpu.VMEM((CHUNK, d), jnp.float32) for _ in range(NBUF)],
            pltpu.VMEM_SHARED((n_pad, d), jnp.float32),
            [pltpu.SemaphoreType.DMA for _ in range(NBUF)],
            [pltpu.SemaphoreType.DMA for _ in range(NBUF)],
        ],
    )(ht, src, dst, norm)


def _tc_in(x, w0, b0, w1, blk):
    n, d = x.shape

    def body(x_ref, w0_ref, b0_ref, w1_ref, h0_ref, ht1_ref):
        h0 = jnp.maximum(
            jnp.dot(x_ref[...], w0_ref[...],
                    preferred_element_type=jnp.float32) + b0_ref[...], 0.0)
        h0_ref[...] = h0
        ht1_ref[...] = jnp.dot(h0, w1_ref[...],
                               preferred_element_type=jnp.float32)

    return pl.pallas_call(
        body,
        grid=(n // blk,),
        in_specs=[
            pl.BlockSpec((blk, d), lambda i: (i, 0)),
            pl.BlockSpec((d, d), lambda i: (0, 0)),
            pl.BlockSpec((1, d), lambda i: (0, 0)),
            pl.BlockSpec((d, d), lambda i: (0, 0)),
        ],
        out_specs=[pl.BlockSpec((blk, d), lambda i: (i, 0))] * 2,
        out_shape=[jax.ShapeDtypeStruct((n, d), jnp.float32)] * 2,
    )(x, w0, b0.reshape(1, d), w1)


def _tc_mid(h0, aggp, b1, w2, blk):
    n, d = h0.shape

    def body(h0_ref, a0_ref, a1_ref, b1_ref, w2_ref, h1_ref, ht2_ref):
        g = jnp.maximum(a0_ref[0] + a1_ref[0] + b1_ref[...], 0.0)
        h1 = h0_ref[...] + g
        h1_ref[...] = h1
        ht2_ref[...] = jnp.dot(h1, w2_ref[...],
                               preferred_element_type=jnp.float32)

    return pl.pallas_call(
        body,
        grid=(n // blk,),
        in_specs=[
            pl.BlockSpec((blk, d), lambda i: (i, 0)),
            pl.BlockSpec((1, blk, d), lambda i: (0, i, 0)),
            pl.BlockSpec((1, blk, d), lambda i: (1, i, 0)),
            pl.BlockSpec((1, d), lambda i: (0, 0)),
            pl.BlockSpec((d, d), lambda i: (0, 0)),
        ],
        out_specs=[pl.BlockSpec((blk, d), lambda i: (i, 0))] * 2,
        out_shape=[jax.ShapeDtypeStruct((n, d), jnp.float32)] * 2,
    )(h0, aggp, aggp, b1.reshape(1, d), w2)


def _tc_out(h1, aggp, b2, blk):
    n, d = h1.shape

    def body(h1_ref, a0_ref, a1_ref, b2_ref, o_ref):
        g = jnp.maximum(a0_ref[0] + a1_ref[0] + b2_ref[...], 0.0)
        o_ref[...] = h1_ref[...] + g

    return pl.pallas_call(
        body,
        grid=(n // blk,),
        in_specs=[
            pl.BlockSpec((blk, d), lambda i: (i, 0)),
            pl.BlockSpec((1, blk, d), lambda i: (0, i, 0)),
            pl.BlockSpec((1, blk, d), lambda i: (1, i, 0)),
            pl.BlockSpec((1, d), lambda i: (0, 0)),
        ],
        out_specs=pl.BlockSpec((blk, d), lambda i: (i, 0)),
        out_shape=jax.ShapeDtypeStruct((n, d), jnp.float32),
    )(h1, aggp, aggp, b2.reshape(1, d))


def kernel(x, edge_index, edge_weight, edge_attr, W0, b0, W1, b1, W2, b2):
    del edge_attr
    n, d = x.shape
    e = edge_index.shape[1]

    src = edge_index[0].astype(jnp.int32)
    dst = edge_index[1].astype(jnp.int32)
    ew = edge_weight.astype(jnp.float32)

    grain = NC * NS * NBUF * CHUNK
    e_pad = -(-e // grain) * grain
    npad = e_pad - e
    if npad:
        pidx = (jnp.arange(npad, dtype=jnp.int32) * 97) % n
        src = jnp.concatenate([src, pidx])
        dst = jnp.concatenate([dst, pidx])
        ew = jnp.concatenate([ew, jnp.zeros((npad,), jnp.float32)])

    n_pad = -(-n // (NS * NL)) * (NS * NL)

    blk = 1000 if n % 1000 == 0 else 8

    degp = _sc_deg(dst, ew, n_pad, e_pad)
    norm = _sc_norm(degp, src, dst, ew, n_pad, e_pad)

    h0, ht1 = _tc_in(x, W0, b0, W1, blk)
    agg1 = _sc_agg(ht1, src, dst, norm, n_pad, e_pad)
    h1, ht2 = _tc_mid(h0, agg1, b1, W2, blk)
    agg2 = _sc_agg(ht2, src, dst, norm, n_pad, e_pad)
    return _tc_out(h1, agg2, b2, blk)


# speedup vs baseline: 22.9781x; 1.0050x over previous
import dataclasses
import functools

import jax
import jax.numpy as jnp
from jax import lax
from jax.experimental import pallas as pl
from jax.experimental.pallas import tpu as pltpu
from jax.experimental.pallas import tpu_sc as plsc

NC = 2
NS = 16
NL = 16
CHUNK = 80
NBUF = 3

_MESH = functools.partial(
    plsc.VectorSubcoreMesh, core_axis_name="c", subcore_axis_name="s"
)


def _sc_params():
    cp = pltpu.CompilerParams()
    if "needs_layout_passes" in pltpu.CompilerParams.__dataclass_fields__:
        cp = dataclasses.replace(cp, needs_layout_passes=False)
    return cp


def _rsqrt16(d):
    i = plsc.bitcast(d, jnp.int32)
    i = jnp.int32(0x5F3759DF) - (i >> 1)
    y = plsc.bitcast(i, jnp.float32)
    for _ in range(3):
        y = y * (1.5 - 0.5 * d * y * y)
    return jnp.where(d > 0, y, 0.0)


def _splat16(vals_ref, e):
    return plsc.load_gather(vals_ref, [jnp.full((16,), e, jnp.int32)])


def _tree_add(vs):
    while len(vs) > 1:
        vs = [a + b for a, b in zip(vs[::2], vs[1::2])]
    return vs[0]


def _sc_deg(dst, ew, n_pad, e_pad):
    nh = n_pad // 2
    npc = n_pad // NS
    epw = e_pad // (NC * NS)
    nchunks = epw // CHUNK

    def body(dst_hbm, ew_hbm, degp_hbm, d2_v, w2_v, h2, hist_v, seg_v,
             acc_v, slab_sh):
        cid = lax.axis_index("c")
        sid = lax.axis_index("s")
        wid = cid * NS + sid
        iota = lax.iota(jnp.int32, 16)
        zero16 = jnp.zeros((16,), jnp.float32)

        pltpu.sync_copy(dst_hbm.at[pl.ds(wid * epw, epw)], d2_v)
        pltpu.sync_copy(ew_hbm.at[pl.ds(wid * epw, epw)], w2_v)

        for p in range(2):
            lo = p * nh

            @pl.loop(0, nh // 16)
            def _(j):
                for r in range(16):
                    h2[r, pl.ds(j * 16, 16)] = zero16

            @pl.loop(0, epw // 16)
            def _(k):
                d16 = d2_v[pl.ds(k * 16, 16)] - lo
                w16 = w2_v[pl.ds(k * 16, 16)]
                msk = (d16 >= 0) & (d16 < nh)
                dc = jnp.minimum(jnp.maximum(d16, 0), nh - 1)
                plsc.addupdate_scatter(h2, [iota, dc], w16, mask=msk)

            @pl.loop(0, nh // 16)
            def _(j):
                vs = [h2[r, pl.ds(j * 16, 16)] for r in range(16)]
                hist_v[pl.ds(lo + j * 16, 16)] = _tree_add(vs)

        pltpu.sync_copy(hist_v, slab_sh.at[sid])
        plsc.subcore_barrier()

        @pl.loop(0, npc // 16)
        def _(j):
            acc_v[pl.ds(j * 16, 16)] = zero16

        for r in range(NS):
            pltpu.sync_copy(slab_sh.at[r, pl.ds(sid * npc, npc)], seg_v)

            @pl.loop(0, npc // 16)
            def _(j):
                sl = pl.ds(j * 16, 16)
                acc_v[sl] = acc_v[sl] + seg_v[sl]

        pltpu.sync_copy(acc_v, degp_hbm.at[cid, pl.ds(sid * npc, npc)])

    return pl.kernel(
        body,
        out_type=jax.ShapeDtypeStruct((NC, n_pad), jnp.float32),
        mesh=_MESH(),
        compiler_params=_sc_params(),
        scratch_types=[
            pltpu.VMEM((epw,), jnp.int32),
            pltpu.VMEM((epw,), jnp.float32),
            pltpu.VMEM((16, n_pad // 2), jnp.float32),
            pltpu.VMEM((n_pad,), jnp.float32),
            pltpu.VMEM((n_pad // NS,), jnp.float32),
            pltpu.VMEM((n_pad // NS,), jnp.float32),
            pltpu.VMEM_SHARED((NS, n_pad), jnp.float32),
        ],
    )(dst, ew)


def _sc_norm(degp, src, dst, ew, n_pad, e_pad):
    epw = e_pad // (NC * NS)
    nchunks = epw // CHUNK

    def body(degp_hbm, src_hbm, dst_hbm, ew_hbm, norm_hbm, deg2_v, dis_v,
             s2_v, d2_v, w2_v, n_v):
        cid = lax.axis_index("c")
        sid = lax.axis_index("s")
        wid = cid * NS + sid
        pltpu.sync_copy(degp_hbm, deg2_v)
        pltpu.sync_copy(src_hbm.at[pl.ds(wid * epw, epw)], s2_v)
        pltpu.sync_copy(dst_hbm.at[pl.ds(wid * epw, epw)], d2_v)
        pltpu.sync_copy(ew_hbm.at[pl.ds(wid * epw, epw)], w2_v)

        @pl.loop(0, n_pad // 16)
        def _(i):
            a = deg2_v[0, pl.ds(i * 16, 16)]
            b = deg2_v[1, pl.ds(i * 16, 16)]
            dis_v[pl.ds(i * 16, 16)] = _rsqrt16(a + b)

        @plsc.parallel_loop(0, epw // 16, 1, unroll=2)
        def _(k):
            sl = pl.ds(k * 16, 16)
            a = plsc.load_gather(dis_v, [s2_v[sl]])
            b = plsc.load_gather(dis_v, [d2_v[sl]])
            n_v[sl] = a * w2_v[sl] * b

        pltpu.sync_copy(n_v, norm_hbm.at[pl.ds(wid * epw, epw)])

    return pl.kernel(
        body,
        out_type=jax.ShapeDtypeStruct((e_pad,), jnp.float32),
        mesh=_MESH(),
        compiler_params=_sc_params(),
        scratch_types=[
            pltpu.VMEM((NC, n_pad), jnp.float32),
            pltpu.VMEM((n_pad,), jnp.float32),
            pltpu.VMEM((epw,), jnp.int32),
            pltpu.VMEM((epw,), jnp.int32),
            pltpu.VMEM((epw,), jnp.float32),
            pltpu.VMEM((epw,), jnp.float32),
        ],
    )(degp, src, dst, ew)


def _sc_agg(ht, src, dst, norm, n_pad, e_pad):
    d = ht.shape[1]
    npc = n_pad // NS
    epw = e_pad // (NC * NS)
    nchunks = epw // CHUNK
    assert nchunks % NBUF == 0 and nchunks >= 2 * NBUF

    def body(ht_hbm, src_hbm, dst_hbm, norm_hbm, aggp_hbm, s2_v, d_v,
             n_v, rows, acc_sh, gsems, ssems):
        cid = lax.axis_index("c")
        sid = lax.axis_index("s")
        wid = cid * NS + sid
        row0 = sid * npc
        ebase = wid * nchunks
        zero16 = jnp.zeros((16,), jnp.float32)

        @pl.loop(0, CHUNK)
        def _(i):
            for k in range(d // 16):
                rows[0][i, pl.ds(k * 16, 16)] = zero16

        for t in range(npc // CHUNK):
            pltpu.sync_copy(rows[0],
                            acc_sh.at[pl.ds(row0 + t * CHUNK, CHUNK)])
        plsc.subcore_barrier()

        pltpu.sync_copy(src_hbm.at[pl.ds(wid * epw, epw)], s2_v)

        def fetch(j, b):
            pltpu.async_copy(
                dst_hbm.at[pl.ds((ebase + j) * CHUNK, CHUNK)], d_v[b],
                gsems[b])
            pltpu.async_copy(
                norm_hbm.at[pl.ds((ebase + j) * CHUNK, CHUNK)], n_v[b],
                gsems[b])
            pltpu.async_copy(ht_hbm.at[s2_v.at[pl.ds(j * CHUNK, CHUNK)]],
                             rows[b], gsems[b])

        def wait_fetch(j, b):
            pltpu.make_async_copy(
                dst_hbm.at[pl.ds((ebase + j) * CHUNK, CHUNK)], d_v[b],
                gsems[b]).wait()
            pltpu.make_async_copy(
                norm_hbm.at[pl.ds((ebase + j) * CHUNK, CHUNK)], n_v[b],
                gsems[b]).wait()
            pltpu.make_async_copy(
                ht_hbm.at[s2_v.at[pl.ds(j * CHUNK, CHUNK)]], rows[b],
                gsems[b]).wait()

        def wait_scat(b):
            pltpu.make_async_copy(rows[b], acc_sh.at[d_v[b]],
                                  ssems[b]).wait()

        fetch(0, 0)
        fetch(1, 1)

        @pl.loop(0, nchunks, step=NBUF)
        def _(j0):
            for b in range(NBUF):
                j = j0 + b
                wait_fetch(j, b)

                @plsc.parallel_loop(0, CHUNK, 1, unroll=4)
                def _(e):
                    nsplat = _splat16(n_v[b], e)
                    for k in range(d // 16):
                        sl = pl.ds(k * 16, 16)
                        rows[b][e, sl] = rows[b][e, sl] * nsplat

                pltpu.async_copy(rows[b], acc_sh.at[d_v[b]], ssems[b],
                                 add=True)

                b2 = (b + 2) % NBUF

                @pl.when(j + 2 < nchunks)
                def _():
                    @pl.when(j >= 1)
                    def _():
                        wait_scat(b2)

                    fetch(j + 2, b2)

        wait_scat((nchunks - 3) % NBUF)
        wait_scat((nchunks - 2) % NBUF)
        wait_scat((nchunks - 1) % NBUF)
        plsc.subcore_barrier()

        @pl.loop(0, npc // CHUNK)
        def _(t):
            pltpu.sync_copy(acc_sh.at[pl.ds(row0 + t * CHUNK, CHUNK)],
                            rows[0])
            pltpu.sync_copy(
                rows[0], aggp_hbm.at[cid, pl.ds(row0 + t * CHUNK, CHUNK)])

    return pl.kernel(
        body,
        out_type=jax.ShapeDtypeStruct((NC, n_pad, d), jnp.float32),
        mesh=_MESH(),
        compiler_params=_sc_params(),
        scratch_types=[
            pltpu.VMEM((epw,), jnp.int32),
            [pltpu.VMEM((CHUNK,), jnp.int32) for _ in range(NBUF)],
            [pltpu.VMEM((CHUNK,), jnp.float32) for _ in range(NBUF)],
            [pltpu.VMEM((CHUNK, d), jnp.float32) for _ in range(NBUF)],
            pltpu.VMEM_SHARED((n_pad, d), jnp.float32),
            [pltpu.SemaphoreType.DMA for _ in range(NBUF)],
            [pltpu.SemaphoreType.DMA for _ in range(NBUF)],
        ],
    )(ht, src, dst, norm)


def _tc_in(x, w0, b0, w1, blk):
    n, d = x.shape

    def body(x_ref, w0_ref, b0_ref, w1_ref, h0_ref, ht1_ref):
        h0 = jnp.maximum(
            jnp.dot(x_ref[...], w0_ref[...],
                    preferred_element_type=jnp.float32) + b0_ref[...], 0.0)
        h0_ref[...] = h0
        ht1_ref[...] = jnp.dot(h0, w1_ref[...],
                               preferred_element_type=jnp.float32)

    return pl.pallas_call(
        body,
        grid=(n // blk,),
        in_specs=[
            pl.BlockSpec((blk, d), lambda i: (i, 0)),
            pl.BlockSpec((d, d), lambda i: (0, 0)),
            pl.BlockSpec((1, d), lambda i: (0, 0)),
            pl.BlockSpec((d, d), lambda i: (0, 0)),
        ],
        out_specs=[pl.BlockSpec((blk, d), lambda i: (i, 0))] * 2,
        out_shape=[jax.ShapeDtypeStruct((n, d), jnp.float32)] * 2,
    )(x, w0, b0.reshape(1, d), w1)


def _tc_mid(h0, aggp, b1, w2, blk):
    n, d = h0.shape

    def body(h0_ref, a0_ref, a1_ref, b1_ref, w2_ref, h1_ref, ht2_ref):
        g = jnp.maximum(a0_ref[0] + a1_ref[0] + b1_ref[...], 0.0)
        h1 = h0_ref[...] + g
        h1_ref[...] = h1
        ht2_ref[...] = jnp.dot(h1, w2_ref[...],
                               preferred_element_type=jnp.float32)

    return pl.pallas_call(
        body,
        grid=(n // blk,),
        in_specs=[
            pl.BlockSpec((blk, d), lambda i: (i, 0)),
            pl.BlockSpec((1, blk, d), lambda i: (0, i, 0)),
            pl.BlockSpec((1, blk, d), lambda i: (1, i, 0)),
            pl.BlockSpec((1, d), lambda i: (0, 0)),
            pl.BlockSpec((d, d), lambda i: (0, 0)),
        ],
        out_specs=[pl.BlockSpec((blk, d), lambda i: (i, 0))] * 2,
        out_shape=[jax.ShapeDtypeStruct((n, d), jnp.float32)] * 2,
    )(h0, aggp, aggp, b1.reshape(1, d), w2)


def _tc_out(h1, aggp, b2, blk):
    n, d = h1.shape

    def body(h1_ref, a0_ref, a1_ref, b2_ref, o_ref):
        g = jnp.maximum(a0_ref[0] + a1_ref[0] + b2_ref[...], 0.0)
        o_ref[...] = h1_ref[...] + g

    return pl.pallas_call(
        body,
        grid=(n // blk,),
        in_specs=[
            pl.BlockSpec((blk, d), lambda i: (i, 0)),
            pl.BlockSpec((1, blk, d), lambda i: (0, i, 0)),
            pl.BlockSpec((1, blk, d), lambda i: (1, i, 0)),
            pl.BlockSpec((1, d), lambda i: (0, 0)),
        ],
        out_specs=pl.BlockSpec((blk, d), lambda i: (i, 0)),
        out_shape=jax.ShapeDtypeStruct((n, d), jnp.float32),
    )(h1, aggp, aggp, b2.reshape(1, d))


def kernel(x, edge_index, edge_weight, edge_attr, W0, b0, W1, b1, W2, b2):
    del edge_attr
    n, d = x.shape
    e = edge_index.shape[1]

    src = edge_index[0].astype(jnp.int32)
    dst = edge_index[1].astype(jnp.int32)
    ew = edge_weight.astype(jnp.float32)

    grain = NC * NS * NBUF * CHUNK
    e_pad = -(-e // grain) * grain
    npad = e_pad - e
    if npad:
        pidx = (jnp.arange(npad, dtype=jnp.int32) * 97) % n
        src = jnp.concatenate([src, pidx])
        dst = jnp.concatenate([dst, pidx])
        ew = jnp.concatenate([ew, jnp.zeros((npad,), jnp.float32)])

    n_pad = -(-n // (NS * NL)) * (NS * NL)

    blk = 1000 if n % 1000 == 0 else 8

    degp = _sc_deg(dst, ew, n_pad, e_pad)
    norm = _sc_norm(degp, src, dst, ew, n_pad, e_pad)

    h0, ht1 = _tc_in(x, W0, b0, W1, blk)
    agg1 = _sc_agg(ht1, src, dst, norm, n_pad, e_pad)
    h1, ht2 = _tc_mid(h0, agg1, b1, W2, blk)
    agg2 = _sc_agg(ht2, src, dst, norm, n_pad, e_pad)
    return _tc_out(h1, agg2, b2, blk)


# speedup vs baseline: 23.0913x; 1.0049x over previous
import dataclasses
import functools

import jax
import jax.numpy as jnp
from jax import lax
from jax.experimental import pallas as pl
from jax.experimental.pallas import tpu as pltpu
from jax.experimental.pallas import tpu_sc as plsc

NC = 2
NS = 16
NL = 16
CHUNK = 80
NBUF = 3

_MESH = functools.partial(
    plsc.VectorSubcoreMesh, core_axis_name="c", subcore_axis_name="s"
)


def _sc_params():
    cp = pltpu.CompilerParams()
    if "needs_layout_passes" in pltpu.CompilerParams.__dataclass_fields__:
        cp = dataclasses.replace(cp, needs_layout_passes=False)
    return cp


def _rsqrt16(d):
    i = plsc.bitcast(d, jnp.int32)
    i = jnp.int32(0x5F3759DF) - (i >> 1)
    y = plsc.bitcast(i, jnp.float32)
    for _ in range(3):
        y = y * (1.5 - 0.5 * d * y * y)
    return jnp.where(d > 0, y, 0.0)


def _splat16(vals_ref, e):
    return plsc.load_gather(vals_ref, [jnp.full((16,), e, jnp.int32)])


def _tree_add(vs):
    while len(vs) > 1:
        vs = [a + b for a, b in zip(vs[::2], vs[1::2])]
    return vs[0]


def _sc_deg(dst, ew, n_pad, e_pad):
    nh = n_pad // 2
    npc = n_pad // NS
    epw = e_pad // (NC * NS)
    nchunks = epw // CHUNK

    def body(dst_hbm, ew_hbm, degp_hbm, d2_v, w2_v, h2, hist_v, seg_v,
             acc_v, slab_sh):
        cid = lax.axis_index("c")
        sid = lax.axis_index("s")
        wid = cid * NS + sid
        iota = lax.iota(jnp.int32, 16)
        zero16 = jnp.zeros((16,), jnp.float32)

        pltpu.sync_copy(dst_hbm.at[pl.ds(wid * epw, epw)], d2_v)
        pltpu.sync_copy(ew_hbm.at[pl.ds(wid * epw, epw)], w2_v)

        for p in range(2):
            lo = p * nh

            @pl.loop(0, nh // 16)
            def _(j):
                for r in range(16):
                    h2[r, pl.ds(j * 16, 16)] = zero16

            @pl.loop(0, epw // 16)
            def _(k):
                d16 = d2_v[pl.ds(k * 16, 16)] - lo
                w16 = w2_v[pl.ds(k * 16, 16)]
                msk = (d16 >= 0) & (d16 < nh)
                dc = jnp.minimum(jnp.maximum(d16, 0), nh - 1)
                plsc.addupdate_scatter(h2, [iota, dc], w16, mask=msk)

            @pl.loop(0, nh // 16)
            def _(j):
                vs = [h2[r, pl.ds(j * 16, 16)] for r in range(16)]
                hist_v[pl.ds(lo + j * 16, 16)] = _tree_add(vs)

        pltpu.sync_copy(hist_v, slab_sh.at[sid])
        plsc.subcore_barrier()

        @pl.loop(0, npc // 16)
        def _(j):
            acc_v[pl.ds(j * 16, 16)] = zero16

        for r in range(NS):
            pltpu.sync_copy(slab_sh.at[r, pl.ds(sid * npc, npc)], seg_v)

            @pl.loop(0, npc // 16)
            def _(j):
                sl = pl.ds(j * 16, 16)
                acc_v[sl] = acc_v[sl] + seg_v[sl]

        pltpu.sync_copy(acc_v, degp_hbm.at[cid, pl.ds(sid * npc, npc)])

    return pl.kernel(
        body,
        out_type=jax.ShapeDtypeStruct((NC, n_pad), jnp.float32),
        mesh=_MESH(),
        compiler_params=_sc_params(),
        scratch_types=[
            pltpu.VMEM((epw,), jnp.int32),
            pltpu.VMEM((epw,), jnp.float32),
            pltpu.VMEM((16, n_pad // 2), jnp.float32),
            pltpu.VMEM((n_pad,), jnp.float32),
            pltpu.VMEM((n_pad // NS,), jnp.float32),
            pltpu.VMEM((n_pad // NS,), jnp.float32),
            pltpu.VMEM_SHARED((NS, n_pad), jnp.float32),
        ],
    )(dst, ew)


def _sc_norm(degp, src, dst, ew, n_pad, e_pad):
    epw = e_pad // (NC * NS)
    nchunks = epw // CHUNK

    def body(degp_hbm, src_hbm, dst_hbm, ew_hbm, norm_hbm, deg2_v, dis_v,
             s2_v, d2_v, w2_v, n_v):
        cid = lax.axis_index("c")
        sid = lax.axis_index("s")
        wid = cid * NS + sid
        pltpu.sync_copy(degp_hbm, deg2_v)
        pltpu.sync_copy(src_hbm.at[pl.ds(wid * epw, epw)], s2_v)
        pltpu.sync_copy(dst_hbm.at[pl.ds(wid * epw, epw)], d2_v)
        pltpu.sync_copy(ew_hbm.at[pl.ds(wid * epw, epw)], w2_v)

        @pl.loop(0, n_pad // 16)
        def _(i):
            a = deg2_v[0, pl.ds(i * 16, 16)]
            b = deg2_v[1, pl.ds(i * 16, 16)]
            dis_v[pl.ds(i * 16, 16)] = _rsqrt16(a + b)

        @plsc.parallel_loop(0, epw // 16, 1, unroll=2)
        def _(k):
            sl = pl.ds(k * 16, 16)
            a = plsc.load_gather(dis_v, [s2_v[sl]])
            b = plsc.load_gather(dis_v, [d2_v[sl]])
            n_v[sl] = a * w2_v[sl] * b

        pltpu.sync_copy(n_v, norm_hbm.at[pl.ds(wid * epw, epw)])

    return pl.kernel(
        body,
        out_type=jax.ShapeDtypeStruct((e_pad,), jnp.float32),
        mesh=_MESH(),
        compiler_params=_sc_params(),
        scratch_types=[
            pltpu.VMEM((NC, n_pad), jnp.float32),
            pltpu.VMEM((n_pad,), jnp.float32),
            pltpu.VMEM((epw,), jnp.int32),
            pltpu.VMEM((epw,), jnp.int32),
            pltpu.VMEM((epw,), jnp.float32),
            pltpu.VMEM((epw,), jnp.float32),
        ],
    )(degp, src, dst, ew)


def _sc_agg(ht, src, dst, norm, n_pad, e_pad):
    d = ht.shape[1]
    npc = n_pad // NS
    epw = e_pad // (NC * NS)
    nchunks = epw // CHUNK
    assert nchunks % NBUF == 0 and nchunks >= 2 * NBUF

    def body(ht_hbm, src_hbm, dst_hbm, norm_hbm, aggp_hbm, s2_v, d_v,
             n_v, rows, acc_sh, gsems, ssems):
        cid = lax.axis_index("c")
        sid = lax.axis_index("s")
        wid = cid * NS + sid
        row0 = sid * npc
        ebase = wid * nchunks
        zero16 = jnp.zeros((16,), jnp.float32)

        @pl.loop(0, CHUNK)
        def _(i):
            for k in range(d // 16):
                rows[0][i, pl.ds(k * 16, 16)] = zero16

        for t in range(npc // CHUNK):
            pltpu.sync_copy(rows[0],
                            acc_sh.at[pl.ds(row0 + t * CHUNK, CHUNK)])
        plsc.subcore_barrier()

        pltpu.sync_copy(src_hbm.at[pl.ds(wid * epw, epw)], s2_v)

        def fetch(j, b):
            pltpu.async_copy(
                dst_hbm.at[pl.ds((ebase + j) * CHUNK, CHUNK)], d_v[b],
                gsems[b])
            pltpu.async_copy(
                norm_hbm.at[pl.ds((ebase + j) * CHUNK, CHUNK)], n_v[b],
                gsems[b])
            pltpu.async_copy(ht_hbm.at[s2_v.at[pl.ds(j * CHUNK, CHUNK)]],
                             rows[b], gsems[b])

        def wait_fetch(j, b):
            pltpu.make_async_copy(
                dst_hbm.at[pl.ds((ebase + j) * CHUNK, CHUNK)], d_v[b],
                gsems[b]).wait()
            pltpu.make_async_copy(
                norm_hbm.at[pl.ds((ebase + j) * CHUNK, CHUNK)], n_v[b],
                gsems[b]).wait()
            pltpu.make_async_copy(
                ht_hbm.at[s2_v.at[pl.ds(j * CHUNK, CHUNK)]], rows[b],
                gsems[b]).wait()

        def wait_scat(b):
            pltpu.make_async_copy(rows[b], acc_sh.at[d_v[b]],
                                  ssems[b]).wait()

        fetch(0, 0)
        fetch(1, 1)

        @pl.loop(0, nchunks, step=NBUF)
        def _(j0):
            for b in range(NBUF):
                j = j0 + b
                wait_fetch(j, b)

                @plsc.parallel_loop(0, CHUNK, 1, unroll=4)
                def _(e):
                    nsplat = _splat16(n_v[b], e)
                    for k in range(d // 16):
                        sl = pl.ds(k * 16, 16)
                        rows[b][e, sl] = rows[b][e, sl] * nsplat

                pltpu.async_copy(rows[b], acc_sh.at[d_v[b]], ssems[b],
                                 add=True)

                b2 = (b + 2) % NBUF

                @pl.when(j + 2 < nchunks)
                def _():
                    @pl.when(j >= 1)
                    def _():
                        wait_scat(b2)

                    fetch(j + 2, b2)

        wait_scat((nchunks - 3) % NBUF)
        wait_scat((nchunks - 2) % NBUF)
        wait_scat((nchunks - 1) % NBUF)
        plsc.subcore_barrier()

        pltpu.sync_copy(acc_sh.at[pl.ds(row0, npc)],
                        aggp_hbm.at[cid, pl.ds(row0, npc)])

    return pl.kernel(
        body,
        out_type=jax.ShapeDtypeStruct((NC, n_pad, d), jnp.float32),
        mesh=_MESH(),
        compiler_params=_sc_params(),
        scratch_types=[
            pltpu.VMEM((epw,), jnp.int32),
            [pltpu.VMEM((CHUNK,), jnp.int32) for _ in range(NBUF)],
            [pltpu.VMEM((CHUNK,), jnp.float32) for _ in range(NBUF)],
            [pltpu.VMEM((CHUNK, d), jnp.float32) for _ in range(NBUF)],
            pltpu.VMEM_SHARED((n_pad, d), jnp.float32),
            [pltpu.SemaphoreType.DMA for _ in range(NBUF)],
            [pltpu.SemaphoreType.DMA for _ in range(NBUF)],
        ],
    )(ht, src, dst, norm)


def _tc_in(x, w0, b0, w1, blk):
    n, d = x.shape

    def body(x_ref, w0_ref, b0_ref, w1_ref, h0_ref, ht1_ref):
        h0 = jnp.maximum(
            jnp.dot(x_ref[...], w0_ref[...],
                    preferred_element_type=jnp.float32) + b0_ref[...], 0.0)
        h0_ref[...] = h0
        ht1_ref[...] = jnp.dot(h0, w1_ref[...],
                               preferred_element_type=jnp.float32)

    return pl.pallas_call(
        body,
        grid=(n // blk,),
        in_specs=[
            pl.BlockSpec((blk, d), lambda i: (i, 0)),
            pl.BlockSpec((d, d), lambda i: (0, 0)),
            pl.BlockSpec((1, d), lambda i: (0, 0)),
            pl.BlockSpec((d, d), lambda i: (0, 0)),
        ],
        out_specs=[pl.BlockSpec((blk, d), lambda i: (i, 0))] * 2,
        out_shape=[jax.ShapeDtypeStruct((n, d), jnp.float32)] * 2,
    )(x, w0, b0.reshape(1, d), w1)


def _tc_mid(h0, aggp, b1, w2, blk):
    n, d = h0.shape

    def body(h0_ref, a0_ref, a1_ref, b1_ref, w2_ref, h1_ref, ht2_ref):
        g = jnp.maximum(a0_ref[0] + a1_ref[0] + b1_ref[...], 0.0)
        h1 = h0_ref[...] + g
        h1_ref[...] = h1
        ht2_ref[...] = jnp.dot(h1, w2_ref[...],
                               preferred_element_type=jnp.float32)

    return pl.pallas_call(
        body,
        grid=(n // blk,),
        in_specs=[
            pl.BlockSpec((blk, d), lambda i: (i, 0)),
            pl.BlockSpec((1, blk, d), lambda i: (0, i, 0)),
            pl.BlockSpec((1, blk, d), lambda i: (1, i, 0)),
            pl.BlockSpec((1, d), lambda i: (0, 0)),
            pl.BlockSpec((d, d), lambda i: (0, 0)),
        ],
        out_specs=[pl.BlockSpec((blk, d), lambda i: (i, 0))] * 2,
        out_shape=[jax.ShapeDtypeStruct((n, d), jnp.float32)] * 2,
    )(h0, aggp, aggp, b1.reshape(1, d), w2)


def _tc_out(h1, aggp, b2, blk):
    n, d = h1.shape

    def body(h1_ref, a0_ref, a1_ref, b2_ref, o_ref):
        g = jnp.maximum(a0_ref[0] + a1_ref[0] + b2_ref[...], 0.0)
        o_ref[...] = h1_ref[...] + g

    return pl.pallas_call(
        body,
        grid=(n // blk,),
        in_specs=[
            pl.BlockSpec((blk, d), lambda i: (i, 0)),
            pl.BlockSpec((1, blk, d), lambda i: (0, i, 0)),
            pl.BlockSpec((1, blk, d), lambda i: (1, i, 0)),
            pl.BlockSpec((1, d), lambda i: (0, 0)),
        ],
        out_specs=pl.BlockSpec((blk, d), lambda i: (i, 0)),
        out_shape=jax.ShapeDtypeStruct((n, d), jnp.float32),
    )(h1, aggp, aggp, b2.reshape(1, d))


def kernel(x, edge_index, edge_weight, edge_attr, W0, b0, W1, b1, W2, b2):
    del edge_attr
    n, d = x.shape
    e = edge_index.shape[1]

    src = edge_index[0].astype(jnp.int32)
    dst = edge_index[1].astype(jnp.int32)
    ew = edge_weight.astype(jnp.float32)

    grain = NC * NS * NBUF * CHUNK
    e_pad = -(-e // grain) * grain
    npad = e_pad - e
    if npad:
        pidx = (jnp.arange(npad, dtype=jnp.int32) * 97) % n
        src = jnp.concatenate([src, pidx])
        dst = jnp.concatenate([dst, pidx])
        ew = jnp.concatenate([ew, jnp.zeros((npad,), jnp.float32)])

    n_pad = -(-n // (NS * NL)) * (NS * NL)

    blk = 1000 if n % 1000 == 0 else 8

    degp = _sc_deg(dst, ew, n_pad, e_pad)
    norm = _sc_norm(degp, src, dst, ew, n_pad, e_pad)

    h0, ht1 = _tc_in(x, W0, b0, W1, blk)
    agg1 = _sc_agg(ht1, src, dst, norm, n_pad, e_pad)
    h1, ht2 = _tc_mid(h0, agg1, b1, W2, blk)
    agg2 = _sc_agg(ht2, src, dst, norm, n_pad, e_pad)
    return _tc_out(h1, agg2, b2, blk)
